# Initial kernel scaffold; baseline (speedup 1.0000x reference)
import functools

import jax
import jax.numpy as jnp
from jax import lax
from jax.experimental import pallas as pl
from jax.experimental.pallas import tpu as pltpu

B, L_SRC, MAX_LEN, D, FILT, NBINS = 16, 512, 2048, 256, 256, 256


def _conv3(h, w_ref, b_ref):
    w0 = w_ref[0]
    w1 = w_ref[1]
    w2 = w_ref[2]
    y0 = jnp.dot(h, w0, preferred_element_type=jnp.float32)
    y1 = jnp.dot(h, w1, preferred_element_type=jnp.float32)
    y2 = jnp.dot(h, w2, preferred_element_type=jnp.float32)
    n = h.shape[0]
    r = lax.broadcasted_iota(jnp.int32, (n, y0.shape[1]), 0)
    y0s = jnp.where(r >= 1, pltpu.roll(y0, 1, 0), 0.0)
    y2s = jnp.where(r <= n - 2, pltpu.roll(y2, -1, 0), 0.0)
    return y1 + y0s + y2s + b_ref[...][None, :]


def _ln(h, g_ref, b_ref):
    mu = jnp.mean(h, axis=-1, keepdims=True)
    var = jnp.mean((h - mu) ** 2, axis=-1, keepdims=True)
    return (h - mu) * lax.rsqrt(var + 1e-5) * g_ref[...][None, :] + b_ref[...][None, :]


def _pred_body(x_ref, w1_ref, b1_ref, g1_ref, be1_ref, w2_ref, b2_ref, g2_ref,
               be2_ref, wl_ref, bl_ref, o_ref):
    h = x_ref[0]
    h = jnp.maximum(_conv3(h, w1_ref, b1_ref), 0.0)
    h = _ln(h, g1_ref, be1_ref)
    h = jnp.maximum(_conv3(h, w2_ref, b2_ref), 0.0)
    h = _ln(h, g2_ref, be2_ref)
    o_ref[0] = jnp.dot(h, wl_ref[...], preferred_element_type=jnp.float32) + bl_ref[0]


def _predictor(x, p):
    full = lambda shape: pl.BlockSpec(shape, lambda b: (0,) * len(shape))
    out = pl.pallas_call(
        _pred_body,
        grid=(B,),
        in_specs=[
            pl.BlockSpec((1, L_SRC, D), lambda b: (b, 0, 0)),
            full((3, D, FILT)), full((FILT,)), full((FILT,)), full((FILT,)),
            full((3, FILT, FILT)), full((FILT,)), full((FILT,)), full((FILT,)),
            full((FILT, 1)), full((1,)),
        ],
        out_specs=pl.BlockSpec((1, L_SRC, 1), lambda b: (b, 0, 0)),
        out_shape=jax.ShapeDtypeStruct((B, L_SRC, 1), jnp.float32),
    )(x, p['w1'], p['b1'], p['g1'], p['be1'], p['w2'], p['b2'], p['g2'],
      p['be2'], p['wl'], p['bl'])
    return out[..., 0]


def _emb_body(x_ref, pt_ref, et_ref, pL_ref, pR_ref, eL_ref, eR_ref,
              pemb_ref, eemb_ref, x1_ref, x2_ref):
    xb = x_ref[0]
    vp = pt_ref[0]
    oh_p = ((vp > pL_ref[...]) & (vp <= pR_ref[...])).astype(jnp.float32)
    x1 = xb + jnp.dot(oh_p, pemb_ref[...], preferred_element_type=jnp.float32)
    ve = et_ref[0]
    oh_e = ((ve > eL_ref[...]) & (ve <= eR_ref[...])).astype(jnp.float32)
    x2 = x1 + jnp.dot(oh_e, eemb_ref[...], preferred_element_type=jnp.float32)
    x1_ref[0] = x1
    x2_ref[0] = x2


def _emb_add(x, pitch_target, energy_target, pitch_bins, energy_bins,
             pitch_emb, energy_emb):
    inf = jnp.array([jnp.inf], jnp.float32)
    pL = jnp.concatenate([-inf, pitch_bins]).reshape(1, NBINS)
    pR = jnp.concatenate([pitch_bins, inf]).reshape(1, NBINS)
    eL = jnp.concatenate([-inf, energy_bins]).reshape(1, NBINS)
    eR = jnp.concatenate([energy_bins, inf]).reshape(1, NBINS)
    pt = pitch_target[..., None]
    et = energy_target[..., None]
    full = lambda shape: pl.BlockSpec(shape, lambda b: (0,) * len(shape))
    x1, x2 = pl.pallas_call(
        _emb_body,
        grid=(B,),
        in_specs=[
            pl.BlockSpec((1, L_SRC, D), lambda b: (b, 0, 0)),
            pl.BlockSpec((1, L_SRC, 1), lambda b: (b, 0, 0)),
            pl.BlockSpec((1, L_SRC, 1), lambda b: (b, 0, 0)),
            full((1, NBINS)), full((1, NBINS)), full((1, NBINS)), full((1, NBINS)),
            full((NBINS, D)), full((NBINS, D)),
        ],
        out_specs=[
            pl.BlockSpec((1, L_SRC, D), lambda b: (b, 0, 0)),
            pl.BlockSpec((1, L_SRC, D), lambda b: (b, 0, 0)),
        ],
        out_shape=[
            jax.ShapeDtypeStruct((B, L_SRC, D), jnp.float32),
            jax.ShapeDtypeStruct((B, L_SRC, D), jnp.float32),
        ],
    )(x, pt, et, pL, pR, eL, eR, pitch_emb, energy_emb)
    return x1, x2


def _lr_body(x2_ref, dur_ref, out_ref, mel_ref):
    d = dur_ref[0].astype(jnp.float32)
    lane = lax.broadcasted_iota(jnp.int32, (1, L_SRC), 1)
    c = d
    s = 1
    while s < L_SRC:
        c = c + jnp.where(lane >= s, pltpu.roll(c, s, 1), 0.0)
        s *= 2
    cum = c
    cumL = jnp.where(lane >= 1, pltpu.roll(cum, 1, 1), 0.0)
    tcol = lax.broadcasted_iota(jnp.float32, (MAX_LEN, L_SRC), 0)
    oh = ((tcol >= cumL) & (tcol < cum)).astype(jnp.float32)
    out_ref[0] = jnp.dot(oh, x2_ref[0], preferred_element_type=jnp.float32)
    mel = lax.slice(cum, (0, L_SRC - 1), (1, L_SRC))
    mel_ref[0] = jnp.broadcast_to(mel.astype(jnp.int32), (1, 128))


def _length_regulate(x2, duration):
    dur3 = duration.reshape(B, 1, L_SRC)
    out, mel = pl.pallas_call(
        _lr_body,
        grid=(B,),
        in_specs=[
            pl.BlockSpec((1, L_SRC, D), lambda b: (b, 0, 0)),
            pl.BlockSpec((1, 1, L_SRC), lambda b: (b, 0, 0)),
        ],
        out_specs=[
            pl.BlockSpec((1, MAX_LEN, D), lambda b: (b, 0, 0)),
            pl.BlockSpec((1, 1, 128), lambda b: (b, 0, 0)),
        ],
        out_shape=[
            jax.ShapeDtypeStruct((B, MAX_LEN, D), jnp.float32),
            jax.ShapeDtypeStruct((B, 1, 128), jnp.int32),
        ],
    )(x2, dur3)
    return out, mel[:, 0, 0]


def kernel(x, src_mask, mel_mask, max_len, pitch_target, energy_target,
           duration_target, dp, pp, ep, pitch_bins, energy_bins,
           pitch_emb, energy_emb):
    log_duration_prediction = _predictor(x, dp)
    pitch_prediction = _predictor(x, pp)
    x1, x2 = _emb_add(x, pitch_target, energy_target, pitch_bins, energy_bins,
                      pitch_emb, energy_emb)
    energy_prediction = _predictor(x1, ep)
    out, mel_len = _length_regulate(x2, duration_target)
    return (out, pitch_prediction, energy_prediction, log_duration_prediction,
            duration_target, mel_len, mel_mask)


# speedup vs baseline: 40.8756x; 40.8756x over previous
import functools

import jax
import jax.numpy as jnp
from jax import lax
from jax.experimental import pallas as pl
from jax.experimental.pallas import tpu as pltpu

B, L_SRC, MAX_LEN, D, FILT, NBINS = 16, 512, 2048, 256, 256, 256


def _conv3(h, w_ref, b_ref):
    w0 = w_ref[0]
    w1 = w_ref[1]
    w2 = w_ref[2]
    y0 = jnp.dot(h, w0, preferred_element_type=jnp.float32)
    y1 = jnp.dot(h, w1, preferred_element_type=jnp.float32)
    y2 = jnp.dot(h, w2, preferred_element_type=jnp.float32)
    n = h.shape[0]
    r = lax.broadcasted_iota(jnp.int32, (n, y0.shape[1]), 0)
    y0s = jnp.where(r >= 1, pltpu.roll(y0, 1, 0), 0.0)
    y2s = jnp.where(r <= n - 2, pltpu.roll(y2, n - 1, 0), 0.0)
    return y1 + y0s + y2s + b_ref[...][None, :]


def _ln(h, g_ref, b_ref):
    mu = jnp.mean(h, axis=-1, keepdims=True)
    var = jnp.mean((h - mu) ** 2, axis=-1, keepdims=True)
    return (h - mu) * lax.rsqrt(var + 1e-5) * g_ref[...][None, :] + b_ref[...][None, :]


def _pred_body(x_ref, w1_ref, b1_ref, g1_ref, be1_ref, w2_ref, b2_ref, g2_ref,
               be2_ref, wl_ref, bl_ref, o_ref):
    h = x_ref[0]
    h = jnp.maximum(_conv3(h, w1_ref, b1_ref), 0.0)
    h = _ln(h, g1_ref, be1_ref)
    h = jnp.maximum(_conv3(h, w2_ref, b2_ref), 0.0)
    h = _ln(h, g2_ref, be2_ref)
    o_ref[0] = jnp.dot(h, wl_ref[...], preferred_element_type=jnp.float32) + bl_ref[0]


def _predictor(x, p):
    full = lambda shape: pl.BlockSpec(shape, lambda b: (0,) * len(shape))
    out = pl.pallas_call(
        _pred_body,
        grid=(B,),
        in_specs=[
            pl.BlockSpec((1, L_SRC, D), lambda b: (b, 0, 0)),
            full((3, D, FILT)), full((FILT,)), full((FILT,)), full((FILT,)),
            full((3, FILT, FILT)), full((FILT,)), full((FILT,)), full((FILT,)),
            full((FILT, 1)), full((1,)),
        ],
        out_specs=pl.BlockSpec((1, L_SRC, 1), lambda b: (b, 0, 0)),
        out_shape=jax.ShapeDtypeStruct((B, L_SRC, 1), jnp.float32),
    )(x, p['w1'], p['b1'], p['g1'], p['be1'], p['w2'], p['b2'], p['g2'],
      p['be2'], p['wl'], p['bl'])
    return out[..., 0]


def _emb_body(x_ref, pt_ref, et_ref, pL_ref, pR_ref, eL_ref, eR_ref,
              pemb_ref, eemb_ref, x1_ref, x2_ref):
    xb = x_ref[0]
    vp = pt_ref[0]
    oh_p = ((vp > pL_ref[...]) & (vp <= pR_ref[...])).astype(jnp.float32)
    x1 = xb + jnp.dot(oh_p, pemb_ref[...], preferred_element_type=jnp.float32)
    ve = et_ref[0]
    oh_e = ((ve > eL_ref[...]) & (ve <= eR_ref[...])).astype(jnp.float32)
    x2 = x1 + jnp.dot(oh_e, eemb_ref[...], preferred_element_type=jnp.float32)
    x1_ref[0] = x1
    x2_ref[0] = x2


def _emb_add(x, pitch_target, energy_target, pitch_bins, energy_bins,
             pitch_emb, energy_emb):
    inf = jnp.array([jnp.inf], jnp.float32)
    pL = jnp.concatenate([-inf, pitch_bins]).reshape(1, NBINS)
    pR = jnp.concatenate([pitch_bins, inf]).reshape(1, NBINS)
    eL = jnp.concatenate([-inf, energy_bins]).reshape(1, NBINS)
    eR = jnp.concatenate([energy_bins, inf]).reshape(1, NBINS)
    pt = pitch_target[..., None]
    et = energy_target[..., None]
    full = lambda shape: pl.BlockSpec(shape, lambda b: (0,) * len(shape))
    x1, x2 = pl.pallas_call(
        _emb_body,
        grid=(B,),
        in_specs=[
            pl.BlockSpec((1, L_SRC, D), lambda b: (b, 0, 0)),
            pl.BlockSpec((1, L_SRC, 1), lambda b: (b, 0, 0)),
            pl.BlockSpec((1, L_SRC, 1), lambda b: (b, 0, 0)),
            full((1, NBINS)), full((1, NBINS)), full((1, NBINS)), full((1, NBINS)),
            full((NBINS, D)), full((NBINS, D)),
        ],
        out_specs=[
            pl.BlockSpec((1, L_SRC, D), lambda b: (b, 0, 0)),
            pl.BlockSpec((1, L_SRC, D), lambda b: (b, 0, 0)),
        ],
        out_shape=[
            jax.ShapeDtypeStruct((B, L_SRC, D), jnp.float32),
            jax.ShapeDtypeStruct((B, L_SRC, D), jnp.float32),
        ],
    )(x, pt, et, pL, pR, eL, eR, pitch_emb, energy_emb)
    return x1, x2


def _lr_body(x2_ref, dur_ref, out_ref, mel_ref):
    d = dur_ref[0]
    lane = lax.broadcasted_iota(jnp.int32, (1, L_SRC), 1)
    c = d
    s = 1
    while s < L_SRC:
        c = c + jnp.where(lane >= s, pltpu.roll(c, s, 1), 0)
        s *= 2
    cum = c
    cumL = jnp.where(lane >= 1, pltpu.roll(cum, 1, 1), 0)
    tcol = lax.broadcasted_iota(jnp.int32, (MAX_LEN, L_SRC), 0)
    oh = ((tcol >= cumL) & (tcol < cum)).astype(jnp.float32)
    out_ref[0] = jnp.dot(oh, x2_ref[0], preferred_element_type=jnp.float32)
    mel = lax.slice(cum, (0, L_SRC - 1), (1, L_SRC))
    mel_ref[0] = jnp.broadcast_to(mel, (1, 128))


def _length_regulate(x2, duration):
    dur3 = duration.reshape(B, 1, L_SRC)
    out, mel = pl.pallas_call(
        _lr_body,
        grid=(B,),
        in_specs=[
            pl.BlockSpec((1, L_SRC, D), lambda b: (b, 0, 0)),
            pl.BlockSpec((1, 1, L_SRC), lambda b: (b, 0, 0)),
        ],
        out_specs=[
            pl.BlockSpec((1, MAX_LEN, D), lambda b: (b, 0, 0)),
            pl.BlockSpec((1, 1, 128), lambda b: (b, 0, 0)),
        ],
        out_shape=[
            jax.ShapeDtypeStruct((B, MAX_LEN, D), jnp.float32),
            jax.ShapeDtypeStruct((B, 1, 128), jnp.int32),
        ],
    )(x2, dur3)
    return out, mel[:, 0, 0]


def kernel(x, src_mask, mel_mask, max_len, pitch_target, energy_target,
           duration_target, dp, pp, ep, pitch_bins, energy_bins,
           pitch_emb, energy_emb):
    log_duration_prediction = _predictor(x, dp)
    pitch_prediction = _predictor(x, pp)
    x1, x2 = _emb_add(x, pitch_target, energy_target, pitch_bins, energy_bins,
                      pitch_emb, energy_emb)
    energy_prediction = _predictor(x1, ep)
    out, mel_len = _length_regulate(x2, duration_target)
    return (out, pitch_prediction, energy_prediction, log_duration_prediction,
            duration_target, mel_len, mel_mask)


# speedup vs baseline: 56.2396x; 1.3759x over previous
import jax
import jax.numpy as jnp
from jax import lax
from jax.experimental import pallas as pl
from jax.experimental.pallas import tpu as pltpu

B, L_SRC, MAX_LEN, D, FILT, NBINS = 16, 512, 2048, 256, 256, 256


def _conv3(h, w_ref, b_ref):
    y0 = jnp.dot(h, w_ref[0], preferred_element_type=jnp.float32)
    y1 = jnp.dot(h, w_ref[1], preferred_element_type=jnp.float32)
    y2 = jnp.dot(h, w_ref[2], preferred_element_type=jnp.float32)
    n = h.shape[0]
    r = lax.broadcasted_iota(jnp.int32, (n, y0.shape[1]), 0)
    y0s = jnp.where(r >= 1, pltpu.roll(y0, 1, 0), 0.0)
    y2s = jnp.where(r <= n - 2, pltpu.roll(y2, n - 1, 0), 0.0)
    return y1 + y0s + y2s + b_ref[...][None, :]


def _ln(h, g_ref, b_ref):
    mu = jnp.mean(h, axis=-1, keepdims=True)
    var = jnp.mean((h - mu) ** 2, axis=-1, keepdims=True)
    return (h - mu) * lax.rsqrt(var + 1e-5) * g_ref[...][None, :] + b_ref[...][None, :]


def _pred(h, w1, b1, g1, be1, w2, b2, g2, be2, wl, bl):
    h = jnp.maximum(_conv3(h, w1, b1), 0.0)
    h = _ln(h, g1, be1)
    h = jnp.maximum(_conv3(h, w2, b2), 0.0)
    h = _ln(h, g2, be2)
    return jnp.dot(h, wl[...], preferred_element_type=jnp.float32) + bl[0]


def _fused_body(x_ref, pt_ref, et_ref, dur_ref,
                pL_ref, pR_ref, eL_ref, eR_ref, pemb_ref, eemb_ref,
                dw1, db1, dg1, dbe1, dw2, db2, dg2, dbe2, dwl, dbl,
                pw1, pb1, pg1, pbe1, pw2, pb2, pg2, pbe2, pwl, pbl,
                ew1, eb1, eg1, ebe1, ew2, eb2, eg2, ebe2, ewl, ebl,
                duro_ref, pito_ref, eno_ref, out_ref, mel_ref):
    xb = x_ref[0]
    duro_ref[0] = _pred(xb, dw1, db1, dg1, dbe1, dw2, db2, dg2, dbe2, dwl, dbl)
    pito_ref[0] = _pred(xb, pw1, pb1, pg1, pbe1, pw2, pb2, pg2, pbe2, pwl, pbl)

    vp = pt_ref[0]
    oh_p = ((vp > pL_ref[...]) & (vp <= pR_ref[...])).astype(jnp.float32)
    x1 = xb + jnp.dot(oh_p, pemb_ref[...], preferred_element_type=jnp.float32)
    eno_ref[0] = _pred(x1, ew1, eb1, eg1, ebe1, ew2, eb2, eg2, ebe2, ewl, ebl)

    ve = et_ref[0]
    oh_e = ((ve > eL_ref[...]) & (ve <= eR_ref[...])).astype(jnp.float32)
    x2 = x1 + jnp.dot(oh_e, eemb_ref[...], preferred_element_type=jnp.float32)

    d = dur_ref[0]
    lane = lax.broadcasted_iota(jnp.int32, (1, L_SRC), 1)
    c = d
    s = 1
    while s < L_SRC:
        c = c + jnp.where(lane >= s, pltpu.roll(c, s, 1), 0)
        s *= 2
    cum = c
    cumL = jnp.where(lane >= 1, pltpu.roll(cum, 1, 1), 0)
    tcol = lax.broadcasted_iota(jnp.int32, (MAX_LEN, L_SRC), 0)
    oh = ((tcol >= cumL) & (tcol < cum)).astype(jnp.float32)
    out_ref[0] = jnp.dot(oh, x2, preferred_element_type=jnp.float32)
    mel = lax.slice(cum, (0, L_SRC - 1), (1, L_SRC))
    mel_ref[0] = jnp.broadcast_to(mel, (1, 128))


def kernel(x, src_mask, mel_mask, max_len, pitch_target, energy_target,
           duration_target, dp, pp, ep, pitch_bins, energy_bins,
           pitch_emb, energy_emb):
    inf = jnp.array([jnp.inf], jnp.float32)
    pL = jnp.concatenate([-inf, pitch_bins]).reshape(1, NBINS)
    pR = jnp.concatenate([pitch_bins, inf]).reshape(1, NBINS)
    eL = jnp.concatenate([-inf, energy_bins]).reshape(1, NBINS)
    eR = jnp.concatenate([energy_bins, inf]).reshape(1, NBINS)
    pt = pitch_target[..., None]
    et = energy_target[..., None]
    dur3 = duration_target.reshape(B, 1, L_SRC)

    full = lambda shape: pl.BlockSpec(shape, lambda b: (0,) * len(shape))
    perb = lambda shape: pl.BlockSpec((1,) + shape, lambda b: (b,) + (0,) * len(shape))
    pred_specs = [
        full((3, D, FILT)), full((FILT,)), full((FILT,)), full((FILT,)),
        full((3, FILT, FILT)), full((FILT,)), full((FILT,)), full((FILT,)),
        full((FILT, 1)), full((1,)),
    ]
    pvals = lambda p: (p['w1'], p['b1'], p['g1'], p['be1'], p['w2'], p['b2'],
                       p['g2'], p['be2'], p['wl'], p['bl'])

    duro, pito, eno, out, mel = pl.pallas_call(
        _fused_body,
        grid=(B,),
        in_specs=[
            perb((L_SRC, D)), perb((L_SRC, 1)), perb((L_SRC, 1)), perb((1, L_SRC)),
            full((1, NBINS)), full((1, NBINS)), full((1, NBINS)), full((1, NBINS)),
            full((NBINS, D)), full((NBINS, D)),
        ] + pred_specs * 3,
        out_specs=[
            perb((L_SRC, 1)), perb((L_SRC, 1)), perb((L_SRC, 1)),
            perb((MAX_LEN, D)), perb((1, 128)),
        ],
        out_shape=[
            jax.ShapeDtypeStruct((B, L_SRC, 1), jnp.float32),
            jax.ShapeDtypeStruct((B, L_SRC, 1), jnp.float32),
            jax.ShapeDtypeStruct((B, L_SRC, 1), jnp.float32),
            jax.ShapeDtypeStruct((B, MAX_LEN, D), jnp.float32),
            jax.ShapeDtypeStruct((B, 1, 128), jnp.int32),
        ],
    )(x, pt, et, dur3, pL, pR, eL, eR, pitch_emb, energy_emb,
      *pvals(dp), *pvals(pp), *pvals(ep))

    return (out, pito[..., 0], eno[..., 0], duro[..., 0],
            duration_target, mel[:, 0, 0], mel_mask)
